# Initial kernel scaffold; baseline (speedup 1.0000x reference)
import functools

import jax
import jax.numpy as jnp
from jax import lax
from jax.experimental import pallas as pl
from jax.experimental.pallas import tpu as pltpu
from jax.experimental.pallas import tpu_sc as plsc

_D = 32
_CHUNK = 3200


@functools.lru_cache(maxsize=None)
def _build(n_rows):
    info = plsc.get_sparse_core_info()
    nw = info.num_cores * info.num_subcores
    per_w = n_rows // nw
    n_ch = per_w // _CHUNK
    assert per_w % _CHUNK == 0 and n_rows % nw == 0

    mesh = plsc.VectorSubcoreMesh(core_axis_name="c", subcore_axis_name="s")

    def body(table_hbm, idx_hbm, out_hbm, idx_v, rows_v, sem):
        wid = lax.axis_index("s") * info.num_cores + lax.axis_index("c")
        base = wid * per_w

        def step(c, carry):
            off = base + c * _CHUNK
            pltpu.sync_copy(idx_hbm.at[pl.ds(off, _CHUNK)], idx_v)
            pltpu.async_copy(table_hbm.at[idx_v], rows_v, sem).wait()
            pltpu.sync_copy(rows_v, out_hbm.at[pl.ds(off, _CHUNK)])
            return carry

        lax.fori_loop(0, n_ch, step, 0)

    return pl.kernel(
        body,
        out_type=jax.ShapeDtypeStruct((n_rows, _D), jnp.float32),
        mesh=mesh,
        scratch_types=[
            pltpu.VMEM((_CHUNK,), jnp.int32),
            pltpu.VMEM((_CHUNK, _D), jnp.float32),
            pltpu.SemaphoreType.DMA,
        ],
    )


def kernel(x, table):
    b, h = x.shape
    flat = x.reshape(-1)
    out = _build(flat.shape[0])(table, flat)
    return out.reshape(b, h, _D)


# speedup vs baseline: 1.1117x; 1.1117x over previous
import functools

import jax
import jax.numpy as jnp
from jax import lax
from jax.experimental import pallas as pl
from jax.experimental.pallas import tpu as pltpu
from jax.experimental.pallas import tpu_sc as plsc

_D = 32
_CHUNK = 1280
_NBUF = 3


@functools.lru_cache(maxsize=None)
def _build(n_rows):
    info = plsc.get_sparse_core_info()
    nw = info.num_cores * info.num_subcores
    per_w = n_rows // nw
    n_ch = per_w // _CHUNK
    assert per_w % _CHUNK == 0 and n_rows % nw == 0 and n_ch > _NBUF

    mesh = plsc.VectorSubcoreMesh(core_axis_name="c", subcore_axis_name="s")

    def body(table_hbm, idx_hbm, out_hbm,
             i0, i1, i2, r0, r1, r2,
             si0, si1, si2, sg0, sg1, sg2, so0, so1, so2):
        idx_v = [i0, i1, i2]
        rows_v = [r0, r1, r2]
        isem = [si0, si1, si2]
        gsem = [sg0, sg1, sg2]
        osem = [so0, so1, so2]

        wid = lax.axis_index("s") * info.num_cores + lax.axis_index("c")
        base = wid * per_w

        ih, gh, sh = {}, {}, {}

        def load_idx(c):
            b = c % _NBUF
            ih[c] = pltpu.async_copy(
                idx_hbm.at[pl.ds(base + c * _CHUNK, _CHUNK)],
                idx_v[b], isem[b])

        def start_gather(c):
            b = c % _NBUF
            gh[c] = pltpu.async_copy(table_hbm.at[idx_v[b]], rows_v[b],
                                     gsem[b])

        def start_store(c):
            b = c % _NBUF
            sh[c] = pltpu.async_copy(
                rows_v[b], out_hbm.at[pl.ds(base + c * _CHUNK, _CHUNK)],
                osem[b])

        load_idx(0)
        load_idx(1)
        load_idx(2)
        ih[0].wait()
        start_gather(0)
        ih[1].wait()
        start_gather(1)

        for c in range(n_ch):
            gh[c].wait()
            start_store(c)
            if c + 3 < n_ch:
                load_idx(c + 3)
            if c + 2 < n_ch:
                if c >= 1:
                    sh[c - 1].wait()
                ih[c + 2].wait()
                start_gather(c + 2)

        sh[n_ch - 3].wait()
        sh[n_ch - 2].wait()
        sh[n_ch - 1].wait()

    return pl.kernel(
        body,
        out_type=jax.ShapeDtypeStruct((n_rows, _D), jnp.float32),
        mesh=mesh,
        scratch_types=(
            [pltpu.VMEM((_CHUNK,), jnp.int32) for _ in range(_NBUF)]
            + [pltpu.VMEM((_CHUNK, _D), jnp.float32) for _ in range(_NBUF)]
            + [pltpu.SemaphoreType.DMA for _ in range(3 * _NBUF)]
        ),
        compiler_params=pltpu.CompilerParams(use_tc_tiling_on_sc=False),
    )


def kernel(x, table):
    b, h = x.shape
    flat = x.reshape(-1)
    out = _build(flat.shape[0])(table, flat)
    return out.reshape(b, h, _D)


# speedup vs baseline: 1.1122x; 1.0005x over previous
import functools

import jax
import jax.numpy as jnp
from jax import lax
from jax.experimental import pallas as pl
from jax.experimental.pallas import tpu as pltpu
from jax.experimental.pallas import tpu_sc as plsc

_D = 32
_CHUNK = 1280
_NBUF = 3


@functools.lru_cache(maxsize=None)
def _build(n_rows):
    info = plsc.get_sparse_core_info()
    nw = info.num_cores * info.num_subcores
    per_w = n_rows // nw
    n_ch = per_w // _CHUNK
    assert per_w % _CHUNK == 0 and n_rows % nw == 0 and n_ch > _NBUF

    mesh = plsc.VectorSubcoreMesh(core_axis_name="c", subcore_axis_name="s")

    def body(table_hbm, idx_hbm, out_hbm,
             i0, i1, i2, r0, r1, r2,
             si0, si1, si2, sg0, sg1, sg2, so0, so1, so2):
        idx_v = [i0, i1, i2]
        rows_v = [r0, r1, r2]
        isem = [si0, si1, si2]
        gsem = [sg0, sg1, sg2]
        osem = [so0, so1, so2]

        wid = lax.axis_index("s") * info.num_cores + lax.axis_index("c")
        base = wid * per_w

        ih, gh, sh = {}, {}, {}

        def load_idx(c):
            b = c % _NBUF
            ih[c] = pltpu.async_copy(
                idx_hbm.at[pl.ds(base + c * _CHUNK, _CHUNK)],
                idx_v[b], isem[b])

        half = _CHUNK // 2

        def start_gather(c):
            b = c % _NBUF
            gh[c] = [
                pltpu.async_copy(
                    table_hbm.at[idx_v[b].at[pl.ds(0, half)]],
                    rows_v[b].at[pl.ds(0, half), :], gsem[b]),
                pltpu.async_copy(
                    table_hbm.at[idx_v[b].at[pl.ds(half, half)]],
                    rows_v[b].at[pl.ds(half, half), :], gsem[b]),
            ]

        def start_store(c):
            b = c % _NBUF
            sh[c] = pltpu.async_copy(
                rows_v[b], out_hbm.at[pl.ds(base + c * _CHUNK, _CHUNK)],
                osem[b])

        load_idx(0)
        load_idx(1)
        load_idx(2)
        ih[0].wait()
        start_gather(0)
        ih[1].wait()
        start_gather(1)

        for c in range(n_ch):
            gh[c][0].wait()
            gh[c][1].wait()
            start_store(c)
            if c + 3 < n_ch:
                load_idx(c + 3)
            if c + 2 < n_ch:
                if c >= 1:
                    sh[c - 1].wait()
                ih[c + 2].wait()
                start_gather(c + 2)

        sh[n_ch - 3].wait()
        sh[n_ch - 2].wait()
        sh[n_ch - 1].wait()

    return pl.kernel(
        body,
        out_type=jax.ShapeDtypeStruct((n_rows, _D), jnp.float32),
        mesh=mesh,
        scratch_types=(
            [pltpu.VMEM((_CHUNK,), jnp.int32) for _ in range(_NBUF)]
            + [pltpu.VMEM((_CHUNK, _D), jnp.float32) for _ in range(_NBUF)]
            + [pltpu.SemaphoreType.DMA for _ in range(3 * _NBUF)]
        ),
        compiler_params=pltpu.CompilerParams(use_tc_tiling_on_sc=False),
    )


def kernel(x, table):
    b, h = x.shape
    flat = x.reshape(-1)
    out = _build(flat.shape[0])(table, flat)
    return out.reshape(b, h, _D)


# speedup vs baseline: 1.1129x; 1.0006x over previous
import functools

import jax
import jax.numpy as jnp
from jax import lax
from jax.experimental import pallas as pl
from jax.experimental.pallas import tpu as pltpu
from jax.experimental.pallas import tpu_sc as plsc

_D = 32
_CHUNK = 1280
_NBUF = 3


@functools.lru_cache(maxsize=None)
def _build(n_rows):
    info = plsc.get_sparse_core_info()
    nw = info.num_cores * info.num_subcores
    per_w = n_rows // nw
    n_ch = per_w // _CHUNK
    assert per_w % _CHUNK == 0 and n_rows % nw == 0 and n_ch > _NBUF

    mesh = plsc.VectorSubcoreMesh(core_axis_name="c", subcore_axis_name="s")

    def body(table_hbm, idx_hbm, out_hbm,
             i0, i1, i2, r0, r1, r2,
             si0, si1, si2, sg0, sg1, sg2, so0, so1, so2):
        idx_v = [i0, i1, i2]
        rows_v = [r0, r1, r2]
        isem = [si0, si1, si2]
        gsem = [sg0, sg1, sg2]
        osem = [so0, so1, so2]

        wid = lax.axis_index("s") * info.num_cores + lax.axis_index("c")
        base = wid * per_w

        ih, gh, sh = {}, {}, {}

        def load_idx(c):
            b = c % _NBUF
            ih[c] = pltpu.async_copy(
                idx_hbm.at[pl.ds(base + c * _CHUNK, _CHUNK)],
                idx_v[b], isem[b])

        def start_gather(c):
            b = c % _NBUF
            gh[c] = pltpu.async_copy(table_hbm.at[idx_v[b]], rows_v[b],
                                     gsem[b])

        def start_store(c):
            b = c % _NBUF
            sh[c] = pltpu.async_copy(
                rows_v[b], out_hbm.at[pl.ds(base + c * _CHUNK, _CHUNK)],
                osem[b])

        load_idx(0)
        load_idx(1)
        load_idx(2)
        ih[0].wait()
        start_gather(0)
        ih[1].wait()
        start_gather(1)

        for c in range(n_ch):
            gh[c].wait()
            start_store(c)
            if c + 3 < n_ch:
                load_idx(c + 3)
            if c + 2 < n_ch:
                if c >= 1:
                    sh[c - 1].wait()
                ih[c + 2].wait()
                start_gather(c + 2)

        sh[n_ch - 3].wait()
        sh[n_ch - 2].wait()
        sh[n_ch - 1].wait()

    return pl.kernel(
        body,
        out_type=jax.ShapeDtypeStruct((n_rows, _D), jnp.float32),
        mesh=mesh,
        scratch_types=(
            [pltpu.VMEM((_CHUNK,), jnp.int32) for _ in range(_NBUF)]
            + [pltpu.VMEM((_CHUNK, _D), jnp.float32) for _ in range(_NBUF)]
            + [pltpu.SemaphoreType.DMA for _ in range(3 * _NBUF)]
        ),
        compiler_params=pltpu.CompilerParams(use_tc_tiling_on_sc=False),
    )


def kernel(x, table):
    b, h = x.shape
    flat = x.reshape(-1)
    out = _build(flat.shape[0])(table, flat)
    return out.reshape(b, h, _D)
